# baseline (speedup 1.0000x reference)
import jax
import jax.numpy as jnp
from jax import lax
from jax.experimental import pallas as pl
from jax.experimental.pallas import tpu as pltpu
from jax.experimental.pallas import tpu_sc as plsc

_NUM_CLASSES = 1000
_N_CTX = 16
_DIM = 512
_SEQ = 77
_SUF = _SEQ - 1 - _N_CTX
_B = 4096

_CTX_W = _N_CTX * _DIM
_SUF_W = _SUF * _DIM
_ROW_W = _SEQ * _DIM

_NC = 2
_NS = 16
_NW = _NC * _NS
_BPW = _B // _NW

_CBA = 2
_CBB = 1
_NA = _BPW // _CBA
_NB = _BPW // _CBB
_PRE_REP = 16


def _sc_body(lab_a_hbm, lab_b_hbm, ctx_hbm, pre_hbm, suf_hbm, out_hbm,
             idx_a, idx_b, idx0, pre_v, buf_a, buf_b,
             sem_a0, sem_a1, sem_b0, sem_b1):
    wid = lax.axis_index("s") * _NC + lax.axis_index("c")
    base = wid * _BPW

    pltpu.sync_copy(lab_a_hbm.at[wid], idx_a)
    pltpu.sync_copy(lab_b_hbm.at[wid], idx_b)

    idx0[pl.ds(0, 16)] = jnp.zeros((16,), jnp.int32)
    pltpu.sync_copy(pre_hbm.at[idx0], pre_v)
    for r in range(_BPW // _PRE_REP):
        pltpu.sync_copy(
            pre_v, out_hbm.at[pl.ds(base + r * _PRE_REP, _PRE_REP),
                              pl.ds(0, _DIM)])

    sems_a = (sem_a0, sem_a1)
    sems_b = (sem_b0, sem_b1)

    def run_pass(table_hbm, idx_ref, buf, sems, n_chunks, cb, col_off, col_w):
        def copy(c, b):
            return pltpu.make_async_copy(
                table_hbm.at[idx_ref.at[c]], buf.at[b], sems[b])

        copy(0, 0).start()
        copy(1, 1).start()

        def body(i, carry):
            for b in range(2):
                cur = 2 * i + b
                copy(cur, b).wait()
                pltpu.sync_copy(
                    buf.at[b],
                    out_hbm.at[pl.ds(base + cur * cb, cb),
                               pl.ds(col_off, col_w)])
                nxt = cur + 2

                @pl.when(nxt < n_chunks)
                def _():
                    copy(nxt, b).start()
            return carry

        lax.fori_loop(0, n_chunks // 2, body, 0)

    run_pass(ctx_hbm, idx_a, buf_a, sems_a, _NA, _CBA, _DIM, _CTX_W)
    run_pass(suf_hbm, idx_b, buf_b, sems_b, _NB, _CBB, _DIM + _CTX_W, _SUF_W)


@jax.jit
def kernel(label, cls_ctx, token_prefix, token_suffix):
    label = label.astype(jnp.int32)
    lab_a = label.reshape(_NW, _NA, _CBA)
    lab_b = label.reshape(_NW, _NB, _CBB)
    ctx2 = cls_ctx.reshape(_NUM_CLASSES, _CTX_W)
    pre2 = token_prefix.reshape(1, _DIM)
    suf2 = token_suffix.reshape(_NUM_CLASSES, _SUF_W)

    mesh = plsc.VectorSubcoreMesh(core_axis_name="c", subcore_axis_name="s")
    out = pl.kernel(
        _sc_body,
        out_type=jax.ShapeDtypeStruct((_B, _ROW_W), jnp.float32),
        mesh=mesh,
        scratch_types=[
            pltpu.VMEM((_NA, _CBA), jnp.int32),
            pltpu.VMEM((_NB, _CBB), jnp.int32),
            pltpu.VMEM((16,), jnp.int32),
            pltpu.VMEM((_PRE_REP, _DIM), jnp.float32),
            pltpu.VMEM((2, _CBA, _CTX_W), jnp.float32),
            pltpu.VMEM((2, _CBB, _SUF_W), jnp.float32),
            pltpu.SemaphoreType.DMA,
            pltpu.SemaphoreType.DMA,
            pltpu.SemaphoreType.DMA,
            pltpu.SemaphoreType.DMA,
        ],
    )(lab_a, lab_b, ctx2, pre2, suf2)
    return out.reshape(_B, _SEQ, _DIM)


# speedup vs baseline: 1.4543x; 1.4543x over previous
import jax
import jax.numpy as jnp
from jax import lax
from jax.experimental import pallas as pl
from jax.experimental.pallas import tpu as pltpu
from jax.experimental.pallas import tpu_sc as plsc

_NUM_CLASSES = 1000
_N_CTX = 16
_DIM = 512
_SEQ = 77
_SUF = _SEQ - 1 - _N_CTX
_B = 4096

_NC = 2
_NS = 16
_NW = _NC * _NS
_BPW = _B // _NW


def _sc_body(lab_hbm, aug_hbm, out_hbm, idx_v, buf, sem0, sem1):
    wid = lax.axis_index("s") * _NC + lax.axis_index("c")
    base = wid * _BPW

    pltpu.sync_copy(lab_hbm.at[wid], idx_v)
    sems = (sem0, sem1)

    def copy(c, b):
        return pltpu.make_async_copy(
            aug_hbm.at[idx_v.at[c]], buf.at[b], sems[b])

    copy(0, 0).start()
    copy(1, 1).start()

    def body(i, carry):
        for b in range(2):
            cur = 2 * i + b
            copy(cur, b).wait()
            pltpu.sync_copy(buf.at[b],
                            out_hbm.at[pl.ds(base + cur, 1), :, :, :])
            copy(jnp.minimum(cur + 2, _BPW - 1), b).start()
        return carry

    lax.fori_loop(0, _BPW // 2, body, 0)
    copy(_BPW - 1, 0).wait()
    copy(_BPW - 1, 1).wait()


@jax.jit
def kernel(label, cls_ctx, token_prefix, token_suffix):
    lab = label.astype(jnp.int32).reshape(_NW, _BPW, 1)
    pre = jnp.broadcast_to(token_prefix, (_NUM_CLASSES, 1, _DIM))
    aug = jnp.concatenate([pre, cls_ctx, token_suffix], axis=1)
    aug4 = aug.reshape(_NUM_CLASSES, _SEQ, 1, _DIM)

    mesh = plsc.VectorSubcoreMesh(core_axis_name="c", subcore_axis_name="s")
    out = pl.kernel(
        _sc_body,
        out_type=jax.ShapeDtypeStruct((_B, _SEQ, 1, _DIM), jnp.float32),
        mesh=mesh,
        scratch_types=[
            pltpu.VMEM((_BPW, 1), jnp.int32),
            pltpu.VMEM((2, 1, _SEQ, 1, _DIM), jnp.float32),
            pltpu.SemaphoreType.DMA,
            pltpu.SemaphoreType.DMA,
        ],
    )(lab, aug4)
    return out.reshape(_B, _SEQ, _DIM)
